# baseline (speedup 1.0000x reference)
import functools

import jax
import jax.numpy as jnp
from jax import lax
from jax.experimental import pallas as pl
from jax.experimental.pallas import tpu as pltpu
from jax.experimental.pallas import tpu_sc as plsc

N = 10000
E = 320000
NPAD = 10112
EPAD = 327680
K = 128
RB = 40
NRB1 = 2
NRB2 = 4
ZROWS = 632
MB = 1000

_mesh = plsc.VectorSubcoreMesh(core_axis_name="c", subcore_axis_name="s")


def _edge_chunks(tab, src, dst, t, sidx, didx, gbuf, acc, sga, sgb, nrb):
    def _gather(k, b, sem):
        return pltpu.make_async_copy(tab.at[sidx.at[k]], gbuf.at[b], sem)

    def outer(r, carry):
        pltpu.sync_copy(src.at[t, r], sidx)
        pltpu.sync_copy(dst.at[t, r], didx)
        _gather(0, 0, sga).start()

        def pair(i, carry2):
            k0 = 2 * i
            k1 = k0 + 1
            _gather(k1, 1, sgb).start()
            _gather(k0, 0, sga).wait()
            pltpu.sync_copy(gbuf.at[0], acc.at[didx.at[k0]], add=True)

            @pl.when(k0 + 2 < RB)
            def _():
                _gather(k0 + 2, 0, sga).start()

            _gather(k1, 1, sgb).wait()
            pltpu.sync_copy(gbuf.at[1], acc.at[didx.at[k1]], add=True)
            return carry2

        lax.fori_loop(0, RB // 2, pair, 0)
        return carry

    lax.fori_loop(0, nrb, outer, 0)


_SC_SCRATCH = [
    pltpu.VMEM((RB, K), jnp.int32),
    pltpu.VMEM((RB, K), jnp.int32),
    pltpu.VMEM((2, K, 128), jnp.float32),
    pltpu.VMEM_SHARED((NPAD, 128), jnp.float32),
    pltpu.SemaphoreType.DMA,
    pltpu.SemaphoreType.DMA,
]


@functools.partial(
    pl.kernel,
    mesh=_mesh,
    out_type=jax.ShapeDtypeStruct((2, NPAD, 128), jnp.float32),
    scratch_types=_SC_SCRATCH,
)
def _agg1(xr, src, dst, zrows, out, sidx, didx, gbuf, acc, sga, sgb):
    c = lax.axis_index("c")
    s = lax.axis_index("s")
    wid = s * 2 + c
    pltpu.sync_copy(zrows, acc.at[pl.ds(s * ZROWS, ZROWS)])
    plsc.subcore_barrier()
    _edge_chunks(xr, src, dst, wid, sidx, didx, gbuf, acc, sga, sgb, NRB1)
    plsc.subcore_barrier()
    pltpu.sync_copy(acc.at[pl.ds(s * ZROWS, ZROWS)],
                    out.at[c, pl.ds(s * ZROWS, ZROWS)])


@functools.partial(
    pl.kernel,
    mesh=_mesh,
    out_type=jax.ShapeDtypeStruct((2, NPAD, 128), jnp.float32),
    scratch_types=_SC_SCRATCH,
)
def _agg2(tl, tr, src, dst, zrows, out, sidx, didx, gbuf, acc, sga, sgb):
    c = lax.axis_index("c")
    s = lax.axis_index("s")
    pltpu.sync_copy(zrows, acc.at[pl.ds(s * ZROWS, ZROWS)])
    plsc.subcore_barrier()

    @pl.when(c == 0)
    def _():
        _edge_chunks(tl, src, dst, s, sidx, didx, gbuf, acc, sga, sgb, NRB2)

    @pl.when(c == 1)
    def _():
        _edge_chunks(tr, src, dst, s, sidx, didx, gbuf, acc, sga, sgb, NRB2)

    plsc.subcore_barrier()
    pltpu.sync_copy(acc.at[pl.ds(s * ZROWS, ZROWS)],
                    out.at[c, pl.ds(s * ZROWS, ZROWS)])


def _relu_body(x_ref, o_ref):
    o_ref[...] = jnp.maximum(x_ref[...], 0.0)


def _mlp1_body(eps_ref, x_ref, p0_ref, p1_ref, wa_ref, ba_ref, wb_ref,
               bb_ref, ol_ref, or_ref):
    e = eps_ref[0, 0]
    t = (1.0 + e) * x_ref[...] + p0_ref[0] + p1_ref[0]
    a = jnp.maximum(
        jnp.dot(t, wa_ref[...], preferred_element_type=jnp.float32)
        + ba_ref[...], 0.0)
    h = jnp.maximum(
        jnp.dot(a, wb_ref[...], preferred_element_type=jnp.float32)
        + bb_ref[...], 0.0)
    ol_ref[...] = h[:, :128]
    or_ref[...] = h[:, 128:]


def _mlp2_body(eps_ref, hl_ref, hr_ref, a0_ref, a1_ref, b_ref, wa_ref,
               ba_ref, wb_ref, bb_ref, wm1_ref, bm1_ref, wm2_ref, bm2_ref,
               o_ref, pool_ref):
    i = pl.program_id(0)
    e = eps_ref[0, 0]
    h1 = jnp.concatenate([hl_ref[...], hr_ref[...]], axis=1)
    agg = jnp.concatenate([a0_ref[0], a1_ref[0]], axis=1)
    t = (1.0 + e) * h1 + agg
    a = jnp.maximum(
        jnp.dot(t, wa_ref[...], preferred_element_type=jnp.float32)
        + ba_ref[...], 0.0)
    h2 = jnp.dot(a, wb_ref[...], preferred_element_type=jnp.float32) + bb_ref[...]
    ids = b_ref[0, 0, :]
    oh = (lax.broadcasted_iota(jnp.int32, (64, MB), 0)
          == ids[None, :]).astype(jnp.float32)
    part = jnp.dot(oh, h2, preferred_element_type=jnp.float32)

    @pl.when(i == 0)
    def _():
        pool_ref[...] = part

    @pl.when(i > 0)
    def _():
        pool_ref[...] += part

    @pl.when(i == pl.num_programs(0) - 1)
    def _():
        q = jnp.maximum(
            jnp.dot(pool_ref[...], wm1_ref[...],
                    preferred_element_type=jnp.float32) + bm1_ref[...], 0.0)
        o_ref[...] = (jnp.dot(q, wm2_ref[...],
                              preferred_element_type=jnp.float32)
                      + bm2_ref[...])


def _full(shape):
    nd = len(shape)

    def im(i):
        return (0,) * nd

    return pl.BlockSpec(shape, im)


def kernel(x, edge_index, batch, eps1, W1a, b1a, W1b, b1b, eps2, W2a, b2a,
           W2b, b2b, Wm1, bm1, Wm2, bm2):
    src = edge_index[0].astype(jnp.int32)
    dst = edge_index[1].astype(jnp.int32)
    pad = EPAD - E
    ppos = jnp.arange(pad, dtype=jnp.int32)
    srcp = jnp.concatenate([src, (ppos * 37) % N])
    dstp = jnp.concatenate([dst, N + ppos % (NPAD - N)])
    src1 = srcp.reshape(32, NRB1, RB, K)
    dst1 = dstp.reshape(32, NRB1, RB, K)
    src2 = srcp.reshape(16, NRB2, RB, K)
    dst2 = dstp.reshape(16, NRB2, RB, K)
    zrows = jnp.zeros((ZROWS, 128), jnp.float32)
    batch3 = batch.astype(jnp.int32).reshape(10, 1, MB)
    e1 = jnp.reshape(eps1, (1, 1))
    e2 = jnp.reshape(eps2, (1, 1))
    b1a_ = b1a.reshape(1, 256)
    b1b_ = b1b.reshape(1, 256)
    b2a_ = b2a.reshape(1, 256)
    b2b_ = b2b.reshape(1, 256)
    bm1_ = bm1.reshape(1, 128)
    bm2_ = bm2.reshape(1, 2)

    xr = pl.pallas_call(
        _relu_body,
        grid=(10,),
        in_specs=[pl.BlockSpec((MB, 128), lambda i: (i, 0))],
        out_specs=pl.BlockSpec((MB, 128), lambda i: (i, 0)),
        out_shape=jax.ShapeDtypeStruct((N, 128), jnp.float32),
    )(x)

    p = _agg1(xr, src1, dst1, zrows)

    hl, hr = pl.pallas_call(
        _mlp1_body,
        grid=(10,),
        in_specs=[
            _full((1, 1)),
            pl.BlockSpec((MB, 128), lambda i: (i, 0)),
            pl.BlockSpec((1, MB, 128), lambda i: (0, i, 0)),
            pl.BlockSpec((1, MB, 128), lambda i: (1, i, 0)),
            _full((128, 256)),
            _full((1, 256)),
            _full((256, 256)),
            _full((1, 256)),
        ],
        out_specs=[
            pl.BlockSpec((MB, 128), lambda i: (i, 0)),
            pl.BlockSpec((MB, 128), lambda i: (i, 0)),
        ],
        out_shape=[
            jax.ShapeDtypeStruct((N, 128), jnp.float32),
            jax.ShapeDtypeStruct((N, 128), jnp.float32),
        ],
    )(e1, x, p, p, W1a, b1a_, W1b, b1b_)

    a2 = _agg2(hl, hr, src2, dst2, zrows)

    out = pl.pallas_call(
        _mlp2_body,
        grid=(10,),
        in_specs=[
            _full((1, 1)),
            pl.BlockSpec((MB, 128), lambda i: (i, 0)),
            pl.BlockSpec((MB, 128), lambda i: (i, 0)),
            pl.BlockSpec((1, MB, 128), lambda i: (0, i, 0)),
            pl.BlockSpec((1, MB, 128), lambda i: (1, i, 0)),
            pl.BlockSpec((1, 1, MB), lambda i: (i, 0, 0)),
            _full((256, 256)),
            _full((1, 256)),
            _full((256, 256)),
            _full((1, 256)),
            _full((256, 128)),
            _full((1, 128)),
            _full((128, 2)),
            _full((1, 2)),
        ],
        out_specs=pl.BlockSpec((64, 2), lambda i: (0, 0)),
        out_shape=jax.ShapeDtypeStruct((64, 2), jnp.float32),
        scratch_shapes=[pltpu.VMEM((64, 256), jnp.float32)],
    )(e2, hl, hr, a2, a2, batch3, W2a, b2a_, W2b, b2b_, Wm1, bm1_, Wm2, bm2_)

    return out


# speedup vs baseline: 1.0068x; 1.0068x over previous
import functools

import jax
import jax.numpy as jnp
from jax import lax
from jax.experimental import pallas as pl
from jax.experimental.pallas import tpu as pltpu
from jax.experimental.pallas import tpu_sc as plsc

N = 10000
E = 320000
NPAD = 10112
EPAD = 327680
K = 128
RB = 40
NRB1 = 2
NRB2 = 4
ZROWS = 632
MB = 1000

_mesh = plsc.VectorSubcoreMesh(core_axis_name="c", subcore_axis_name="s")


def _edge_chunks(tab, src, dst, t, sidx, didx, gbuf, acc, sga, sgb, nrb):
    def _gather(k, b, sem):
        return pltpu.make_async_copy(tab.at[sidx.at[k]], gbuf.at[b], sem)

    def outer(r, carry):
        @pl.when(r > 0)
        def _():
            pltpu.sync_copy(src.at[t, r], sidx)
            pltpu.sync_copy(dst.at[t, r], didx)

        _gather(0, 0, sga).start()

        def pair(i, carry2):
            k0 = 2 * i
            k1 = k0 + 1
            _gather(k1, 1, sgb).start()
            _gather(k0, 0, sga).wait()
            pltpu.sync_copy(gbuf.at[0], acc.at[didx.at[k0]], add=True)

            @pl.when(k0 + 2 < RB)
            def _():
                _gather(k0 + 2, 0, sga).start()

            _gather(k1, 1, sgb).wait()
            pltpu.sync_copy(gbuf.at[1], acc.at[didx.at[k1]], add=True)
            return carry2

        lax.fori_loop(0, RB // 2, pair, 0)
        return carry

    lax.fori_loop(0, nrb, outer, 0)


def _prologue(src, dst, t, sidx, didx, zrows, acc, s, sga, sgb):
    pltpu.make_async_copy(src.at[t, 0], sidx, sga).start()
    pltpu.make_async_copy(dst.at[t, 0], didx, sgb).start()
    pltpu.sync_copy(zrows, acc.at[pl.ds(s * ZROWS, ZROWS)])
    pltpu.make_async_copy(src.at[t, 0], sidx, sga).wait()
    pltpu.make_async_copy(dst.at[t, 0], didx, sgb).wait()
    plsc.subcore_barrier()


_SC_SCRATCH = [
    pltpu.VMEM((RB, K), jnp.int32),
    pltpu.VMEM((RB, K), jnp.int32),
    pltpu.VMEM((2, K, 128), jnp.float32),
    pltpu.VMEM_SHARED((NPAD, 128), jnp.float32),
    pltpu.SemaphoreType.DMA,
    pltpu.SemaphoreType.DMA,
]


@functools.partial(
    pl.kernel,
    mesh=_mesh,
    out_type=jax.ShapeDtypeStruct((2, NPAD, 128), jnp.float32),
    scratch_types=_SC_SCRATCH,
)
def _agg1(xr, src, dst, zrows, out, sidx, didx, gbuf, acc, sga, sgb):
    c = lax.axis_index("c")
    s = lax.axis_index("s")
    wid = s * 2 + c
    _prologue(src, dst, wid, sidx, didx, zrows, acc, s, sga, sgb)
    _edge_chunks(xr, src, dst, wid, sidx, didx, gbuf, acc, sga, sgb, NRB1)
    plsc.subcore_barrier()
    pltpu.sync_copy(acc.at[pl.ds(s * ZROWS, ZROWS)],
                    out.at[c, pl.ds(s * ZROWS, ZROWS)])


@functools.partial(
    pl.kernel,
    mesh=_mesh,
    out_type=jax.ShapeDtypeStruct((2, NPAD, 128), jnp.float32),
    scratch_types=_SC_SCRATCH,
)
def _agg2(tl, tr, src, dst, zrows, out, sidx, didx, gbuf, acc, sga, sgb):
    c = lax.axis_index("c")
    s = lax.axis_index("s")
    _prologue(src, dst, s, sidx, didx, zrows, acc, s, sga, sgb)

    @pl.when(c == 0)
    def _():
        _edge_chunks(tl, src, dst, s, sidx, didx, gbuf, acc, sga, sgb, NRB2)

    @pl.when(c == 1)
    def _():
        _edge_chunks(tr, src, dst, s, sidx, didx, gbuf, acc, sga, sgb, NRB2)

    plsc.subcore_barrier()
    pltpu.sync_copy(acc.at[pl.ds(s * ZROWS, ZROWS)],
                    out.at[c, pl.ds(s * ZROWS, ZROWS)])


def _relu_body(x_ref, o_ref):
    o_ref[...] = jnp.maximum(x_ref[...], 0.0)


def _mlp1_body(eps_ref, x_ref, p0_ref, p1_ref, wa_ref, ba_ref, wb_ref,
               bb_ref, ol_ref, or_ref):
    e = eps_ref[0, 0]
    t = (1.0 + e) * x_ref[...] + p0_ref[0] + p1_ref[0]
    a = jnp.maximum(
        jnp.dot(t, wa_ref[...], preferred_element_type=jnp.float32)
        + ba_ref[...], 0.0)
    h = jnp.maximum(
        jnp.dot(a, wb_ref[...], preferred_element_type=jnp.float32)
        + bb_ref[...], 0.0)
    ol_ref[...] = h[:, :128]
    or_ref[...] = h[:, 128:]


def _mlp2_body(eps_ref, hl_ref, hr_ref, a0_ref, a1_ref, b_ref, wa_ref,
               ba_ref, wb_ref, bb_ref, wm1_ref, bm1_ref, wm2_ref, bm2_ref,
               o_ref, pool_ref):
    i = pl.program_id(0)
    e = eps_ref[0, 0]
    h1 = jnp.concatenate([hl_ref[...], hr_ref[...]], axis=1)
    agg = jnp.concatenate([a0_ref[0], a1_ref[0]], axis=1)
    t = (1.0 + e) * h1 + agg
    a = jnp.maximum(
        jnp.dot(t, wa_ref[...], preferred_element_type=jnp.float32)
        + ba_ref[...], 0.0)
    h2 = jnp.dot(a, wb_ref[...], preferred_element_type=jnp.float32) + bb_ref[...]
    ids = b_ref[0, 0, :]
    oh = (lax.broadcasted_iota(jnp.int32, (64, MB), 0)
          == ids[None, :]).astype(jnp.float32)
    part = jnp.dot(oh, h2, preferred_element_type=jnp.float32)

    @pl.when(i == 0)
    def _():
        pool_ref[...] = part

    @pl.when(i > 0)
    def _():
        pool_ref[...] += part

    @pl.when(i == pl.num_programs(0) - 1)
    def _():
        q = jnp.maximum(
            jnp.dot(pool_ref[...], wm1_ref[...],
                    preferred_element_type=jnp.float32) + bm1_ref[...], 0.0)
        o_ref[...] = (jnp.dot(q, wm2_ref[...],
                              preferred_element_type=jnp.float32)
                      + bm2_ref[...])


def _full(shape):
    nd = len(shape)

    def im(i):
        return (0,) * nd

    return pl.BlockSpec(shape, im)


def kernel(x, edge_index, batch, eps1, W1a, b1a, W1b, b1b, eps2, W2a, b2a,
           W2b, b2b, Wm1, bm1, Wm2, bm2):
    src = edge_index[0].astype(jnp.int32)
    dst = edge_index[1].astype(jnp.int32)
    pad = EPAD - E
    ppos = jnp.arange(pad, dtype=jnp.int32)
    srcp = jnp.concatenate([src, (ppos * 37) % N])
    dstp = jnp.concatenate([dst, N + ppos % (NPAD - N)])
    src1 = srcp.reshape(32, NRB1, RB, K)
    dst1 = dstp.reshape(32, NRB1, RB, K)
    src2 = srcp.reshape(16, NRB2, RB, K)
    dst2 = dstp.reshape(16, NRB2, RB, K)
    zrows = jnp.zeros((ZROWS, 128), jnp.float32)
    batch3 = batch.astype(jnp.int32).reshape(10, 1, MB)
    e1 = jnp.reshape(eps1, (1, 1))
    e2 = jnp.reshape(eps2, (1, 1))
    b1a_ = b1a.reshape(1, 256)
    b1b_ = b1b.reshape(1, 256)
    b2a_ = b2a.reshape(1, 256)
    b2b_ = b2b.reshape(1, 256)
    bm1_ = bm1.reshape(1, 128)
    bm2_ = bm2.reshape(1, 2)

    xr = pl.pallas_call(
        _relu_body,
        grid=(10,),
        in_specs=[pl.BlockSpec((MB, 128), lambda i: (i, 0))],
        out_specs=pl.BlockSpec((MB, 128), lambda i: (i, 0)),
        out_shape=jax.ShapeDtypeStruct((N, 128), jnp.float32),
    )(x)

    p = _agg1(xr, src1, dst1, zrows)

    hl, hr = pl.pallas_call(
        _mlp1_body,
        grid=(10,),
        in_specs=[
            _full((1, 1)),
            pl.BlockSpec((MB, 128), lambda i: (i, 0)),
            pl.BlockSpec((1, MB, 128), lambda i: (0, i, 0)),
            pl.BlockSpec((1, MB, 128), lambda i: (1, i, 0)),
            _full((128, 256)),
            _full((1, 256)),
            _full((256, 256)),
            _full((1, 256)),
        ],
        out_specs=[
            pl.BlockSpec((MB, 128), lambda i: (i, 0)),
            pl.BlockSpec((MB, 128), lambda i: (i, 0)),
        ],
        out_shape=[
            jax.ShapeDtypeStruct((N, 128), jnp.float32),
            jax.ShapeDtypeStruct((N, 128), jnp.float32),
        ],
    )(e1, x, p, p, W1a, b1a_, W1b, b1b_)

    a2 = _agg2(hl, hr, src2, dst2, zrows)

    out = pl.pallas_call(
        _mlp2_body,
        grid=(10,),
        in_specs=[
            _full((1, 1)),
            pl.BlockSpec((MB, 128), lambda i: (i, 0)),
            pl.BlockSpec((MB, 128), lambda i: (i, 0)),
            pl.BlockSpec((1, MB, 128), lambda i: (0, i, 0)),
            pl.BlockSpec((1, MB, 128), lambda i: (1, i, 0)),
            pl.BlockSpec((1, 1, MB), lambda i: (i, 0, 0)),
            _full((256, 256)),
            _full((1, 256)),
            _full((256, 256)),
            _full((1, 256)),
            _full((256, 128)),
            _full((1, 128)),
            _full((128, 2)),
            _full((1, 2)),
        ],
        out_specs=pl.BlockSpec((64, 2), lambda i: (0, 0)),
        out_shape=jax.ShapeDtypeStruct((64, 2), jnp.float32),
        scratch_shapes=[pltpu.VMEM((64, 256), jnp.float32)],
    )(e2, hl, hr, a2, a2, batch3, W2a, b2a_, W2b, b2b_, Wm1, bm1_, Wm2, bm2_)

    return out


# speedup vs baseline: 1.0289x; 1.0219x over previous
import functools

import jax
import jax.numpy as jnp
from jax import lax
from jax.experimental import pallas as pl
from jax.experimental.pallas import tpu as pltpu
from jax.experimental.pallas import tpu_sc as plsc

N = 10000
E = 320000
NPAD = 10112
EPAD = 327680
K = 128
RB = 40
NRB1 = 2
NRB2 = 4
ZROWS = 632
MB = 2000

_mesh = plsc.VectorSubcoreMesh(core_axis_name="c", subcore_axis_name="s")


def _edge_chunks(tab, src, dst, t, sidx, didx, gbuf, acc, sga, sgb, nrb):
    def _gather(k, b, sem):
        return pltpu.make_async_copy(tab.at[sidx.at[k]], gbuf.at[b], sem)

    def outer(r, carry):
        @pl.when(r > 0)
        def _():
            pltpu.sync_copy(src.at[t, r], sidx)
            pltpu.sync_copy(dst.at[t, r], didx)

        _gather(0, 0, sga).start()

        def pair(i, carry2):
            k0 = 2 * i
            k1 = k0 + 1
            _gather(k1, 1, sgb).start()
            _gather(k0, 0, sga).wait()
            pltpu.sync_copy(gbuf.at[0], acc.at[didx.at[k0]], add=True)

            @pl.when(k0 + 2 < RB)
            def _():
                _gather(k0 + 2, 0, sga).start()

            _gather(k1, 1, sgb).wait()
            pltpu.sync_copy(gbuf.at[1], acc.at[didx.at[k1]], add=True)
            return carry2

        lax.fori_loop(0, RB // 2, pair, 0)
        return carry

    lax.fori_loop(0, nrb, outer, 0)


def _prologue(src, dst, t, sidx, didx, zrows, acc, s, sga, sgb):
    pltpu.make_async_copy(src.at[t, 0], sidx, sga).start()
    pltpu.make_async_copy(dst.at[t, 0], didx, sgb).start()
    pltpu.sync_copy(zrows, acc.at[pl.ds(s * ZROWS, ZROWS)])
    pltpu.make_async_copy(src.at[t, 0], sidx, sga).wait()
    pltpu.make_async_copy(dst.at[t, 0], didx, sgb).wait()
    plsc.subcore_barrier()


_SC_SCRATCH = [
    pltpu.VMEM((RB, K), jnp.int32),
    pltpu.VMEM((RB, K), jnp.int32),
    pltpu.VMEM((2, K, 128), jnp.float32),
    pltpu.VMEM_SHARED((NPAD, 128), jnp.float32),
    pltpu.SemaphoreType.DMA,
    pltpu.SemaphoreType.DMA,
]


@functools.partial(
    pl.kernel,
    mesh=_mesh,
    out_type=jax.ShapeDtypeStruct((2, NPAD, 128), jnp.float32),
    scratch_types=_SC_SCRATCH,
)
def _agg1(xr, src, dst, zrows, out, sidx, didx, gbuf, acc, sga, sgb):
    c = lax.axis_index("c")
    s = lax.axis_index("s")
    wid = s * 2 + c
    _prologue(src, dst, wid, sidx, didx, zrows, acc, s, sga, sgb)
    _edge_chunks(xr, src, dst, wid, sidx, didx, gbuf, acc, sga, sgb, NRB1)
    plsc.subcore_barrier()
    pltpu.sync_copy(acc.at[pl.ds(s * ZROWS, ZROWS)],
                    out.at[c, pl.ds(s * ZROWS, ZROWS)])


@functools.partial(
    pl.kernel,
    mesh=_mesh,
    out_type=jax.ShapeDtypeStruct((2, NPAD, 128), jnp.float32),
    scratch_types=_SC_SCRATCH,
)
def _agg2(tl, tr, src, dst, zrows, out, sidx, didx, gbuf, acc, sga, sgb):
    c = lax.axis_index("c")
    s = lax.axis_index("s")
    _prologue(src, dst, s, sidx, didx, zrows, acc, s, sga, sgb)

    @pl.when(c == 0)
    def _():
        _edge_chunks(tl, src, dst, s, sidx, didx, gbuf, acc, sga, sgb, NRB2)

    @pl.when(c == 1)
    def _():
        _edge_chunks(tr, src, dst, s, sidx, didx, gbuf, acc, sga, sgb, NRB2)

    plsc.subcore_barrier()
    pltpu.sync_copy(acc.at[pl.ds(s * ZROWS, ZROWS)],
                    out.at[c, pl.ds(s * ZROWS, ZROWS)])


def _relu_body(x_ref, o_ref):
    o_ref[...] = jnp.maximum(x_ref[...], 0.0)


def _mlp1_body(eps_ref, x_ref, p0_ref, p1_ref, wa_ref, ba_ref, wb_ref,
               bb_ref, ol_ref, or_ref):
    e = eps_ref[0, 0]
    t = (1.0 + e) * x_ref[...] + p0_ref[0] + p1_ref[0]
    a = jnp.maximum(
        jnp.dot(t, wa_ref[...], preferred_element_type=jnp.float32)
        + ba_ref[...], 0.0)
    h = jnp.maximum(
        jnp.dot(a, wb_ref[...], preferred_element_type=jnp.float32)
        + bb_ref[...], 0.0)
    ol_ref[...] = h[:, :128]
    or_ref[...] = h[:, 128:]


def _mlp2_body(eps_ref, hl_ref, hr_ref, a0_ref, a1_ref, b_ref, wa_ref,
               ba_ref, wb_ref, bb_ref, wm1_ref, bm1_ref, wm2_ref, bm2_ref,
               o_ref, pool_ref):
    i = pl.program_id(0)
    e = eps_ref[0, 0]
    h1 = jnp.concatenate([hl_ref[...], hr_ref[...]], axis=1)
    agg = jnp.concatenate([a0_ref[0], a1_ref[0]], axis=1)
    t = (1.0 + e) * h1 + agg
    a = jnp.maximum(
        jnp.dot(t, wa_ref[...], preferred_element_type=jnp.float32)
        + ba_ref[...], 0.0)
    h2 = jnp.dot(a, wb_ref[...], preferred_element_type=jnp.float32) + bb_ref[...]
    ids = b_ref[0, 0, :]
    oh = (lax.broadcasted_iota(jnp.int32, (64, MB), 0)
          == ids[None, :]).astype(jnp.float32)
    part = jnp.dot(oh, h2, preferred_element_type=jnp.float32)

    @pl.when(i == 0)
    def _():
        pool_ref[...] = part

    @pl.when(i > 0)
    def _():
        pool_ref[...] += part

    @pl.when(i == pl.num_programs(0) - 1)
    def _():
        q = jnp.maximum(
            jnp.dot(pool_ref[...], wm1_ref[...],
                    preferred_element_type=jnp.float32) + bm1_ref[...], 0.0)
        o_ref[...] = (jnp.dot(q, wm2_ref[...],
                              preferred_element_type=jnp.float32)
                      + bm2_ref[...])


def _full(shape):
    nd = len(shape)

    def im(i):
        return (0,) * nd

    return pl.BlockSpec(shape, im)


def kernel(x, edge_index, batch, eps1, W1a, b1a, W1b, b1b, eps2, W2a, b2a,
           W2b, b2b, Wm1, bm1, Wm2, bm2):
    src = edge_index[0].astype(jnp.int32)
    dst = edge_index[1].astype(jnp.int32)
    pad = EPAD - E
    ppos = jnp.arange(pad, dtype=jnp.int32)
    srcp = jnp.concatenate([src, (ppos * 37) % N])
    dstp = jnp.concatenate([dst, N + ppos % (NPAD - N)])
    src1 = srcp.reshape(32, NRB1, RB, K)
    dst1 = dstp.reshape(32, NRB1, RB, K)
    src2 = srcp.reshape(16, NRB2, RB, K)
    dst2 = dstp.reshape(16, NRB2, RB, K)
    zrows = jnp.zeros((ZROWS, 128), jnp.float32)
    batch3 = batch.astype(jnp.int32).reshape(5, 1, MB)
    e1 = jnp.reshape(eps1, (1, 1))
    e2 = jnp.reshape(eps2, (1, 1))
    b1a_ = b1a.reshape(1, 256)
    b1b_ = b1b.reshape(1, 256)
    b2a_ = b2a.reshape(1, 256)
    b2b_ = b2b.reshape(1, 256)
    bm1_ = bm1.reshape(1, 128)
    bm2_ = bm2.reshape(1, 2)

    xr = pl.pallas_call(
        _relu_body,
        grid=(5,),
        in_specs=[pl.BlockSpec((MB, 128), lambda i: (i, 0))],
        out_specs=pl.BlockSpec((MB, 128), lambda i: (i, 0)),
        out_shape=jax.ShapeDtypeStruct((N, 128), jnp.float32),
    )(x)

    p = _agg1(xr, src1, dst1, zrows)

    hl, hr = pl.pallas_call(
        _mlp1_body,
        grid=(5,),
        in_specs=[
            _full((1, 1)),
            pl.BlockSpec((MB, 128), lambda i: (i, 0)),
            pl.BlockSpec((1, MB, 128), lambda i: (0, i, 0)),
            pl.BlockSpec((1, MB, 128), lambda i: (1, i, 0)),
            _full((128, 256)),
            _full((1, 256)),
            _full((256, 256)),
            _full((1, 256)),
        ],
        out_specs=[
            pl.BlockSpec((MB, 128), lambda i: (i, 0)),
            pl.BlockSpec((MB, 128), lambda i: (i, 0)),
        ],
        out_shape=[
            jax.ShapeDtypeStruct((N, 128), jnp.float32),
            jax.ShapeDtypeStruct((N, 128), jnp.float32),
        ],
    )(e1, x, p, p, W1a, b1a_, W1b, b1b_)

    a2 = _agg2(hl, hr, src2, dst2, zrows)

    out = pl.pallas_call(
        _mlp2_body,
        grid=(5,),
        in_specs=[
            _full((1, 1)),
            pl.BlockSpec((MB, 128), lambda i: (i, 0)),
            pl.BlockSpec((MB, 128), lambda i: (i, 0)),
            pl.BlockSpec((1, MB, 128), lambda i: (0, i, 0)),
            pl.BlockSpec((1, MB, 128), lambda i: (1, i, 0)),
            pl.BlockSpec((1, 1, MB), lambda i: (i, 0, 0)),
            _full((256, 256)),
            _full((1, 256)),
            _full((256, 256)),
            _full((1, 256)),
            _full((256, 128)),
            _full((1, 128)),
            _full((128, 2)),
            _full((1, 2)),
        ],
        out_specs=pl.BlockSpec((64, 2), lambda i: (0, 0)),
        out_shape=jax.ShapeDtypeStruct((64, 2), jnp.float32),
        scratch_shapes=[pltpu.VMEM((64, 256), jnp.float32)],
    )(e2, hl, hr, a2, a2, batch3, W2a, b2a_, W2b, b2b_, Wm1, bm1_, Wm2, bm2_)

    return out


# speedup vs baseline: 1.0440x; 1.0147x over previous
import functools

import jax
import jax.numpy as jnp
from jax import lax
from jax.experimental import pallas as pl
from jax.experimental.pallas import tpu as pltpu
from jax.experimental.pallas import tpu_sc as plsc

N = 10000
E = 320000
NPAD = 10112
EPAD = 327680
K = 128
RB = 40
NRB1 = 2
NRB2 = 4
ZROWS = 632
MB = 5000

_mesh = plsc.VectorSubcoreMesh(core_axis_name="c", subcore_axis_name="s")


def _edge_chunks(tab, src, dst, t, sidx, didx, gbuf, acc, sga, sgb, nrb):
    def _gather(k, b, sem):
        return pltpu.make_async_copy(tab.at[sidx.at[k]], gbuf.at[b], sem)

    def outer(r, carry):
        @pl.when(r > 0)
        def _():
            pltpu.sync_copy(src.at[t, r], sidx)
            pltpu.sync_copy(dst.at[t, r], didx)

        _gather(0, 0, sga).start()

        def pair(i, carry2):
            k0 = 2 * i
            k1 = k0 + 1
            _gather(k1, 1, sgb).start()
            _gather(k0, 0, sga).wait()
            pltpu.sync_copy(gbuf.at[0], acc.at[didx.at[k0]], add=True)

            @pl.when(k0 + 2 < RB)
            def _():
                _gather(k0 + 2, 0, sga).start()

            _gather(k1, 1, sgb).wait()
            pltpu.sync_copy(gbuf.at[1], acc.at[didx.at[k1]], add=True)
            return carry2

        lax.fori_loop(0, RB // 2, pair, 0)
        return carry

    lax.fori_loop(0, nrb, outer, 0)


def _prologue(src, dst, t, sidx, didx, zrows, acc, s, sga, sgb):
    pltpu.make_async_copy(src.at[t, 0], sidx, sga).start()
    pltpu.make_async_copy(dst.at[t, 0], didx, sgb).start()
    pltpu.sync_copy(zrows, acc.at[pl.ds(s * ZROWS, ZROWS)])
    pltpu.make_async_copy(src.at[t, 0], sidx, sga).wait()
    pltpu.make_async_copy(dst.at[t, 0], didx, sgb).wait()
    plsc.subcore_barrier()


_SC_SCRATCH = [
    pltpu.VMEM((RB, K), jnp.int32),
    pltpu.VMEM((RB, K), jnp.int32),
    pltpu.VMEM((2, K, 128), jnp.float32),
    pltpu.VMEM_SHARED((NPAD, 128), jnp.float32),
    pltpu.SemaphoreType.DMA,
    pltpu.SemaphoreType.DMA,
]


@functools.partial(
    pl.kernel,
    mesh=_mesh,
    out_type=jax.ShapeDtypeStruct((2, NPAD, 128), jnp.float32),
    scratch_types=_SC_SCRATCH,
)
def _agg1(xr, src, dst, zrows, out, sidx, didx, gbuf, acc, sga, sgb):
    c = lax.axis_index("c")
    s = lax.axis_index("s")
    wid = s * 2 + c
    _prologue(src, dst, wid, sidx, didx, zrows, acc, s, sga, sgb)
    _edge_chunks(xr, src, dst, wid, sidx, didx, gbuf, acc, sga, sgb, NRB1)
    plsc.subcore_barrier()
    pltpu.sync_copy(acc.at[pl.ds(s * ZROWS, ZROWS)],
                    out.at[c, pl.ds(s * ZROWS, ZROWS)])


@functools.partial(
    pl.kernel,
    mesh=_mesh,
    out_type=jax.ShapeDtypeStruct((2, NPAD, 128), jnp.float32),
    scratch_types=_SC_SCRATCH,
)
def _agg2(tl, tr, src, dst, zrows, out, sidx, didx, gbuf, acc, sga, sgb):
    c = lax.axis_index("c")
    s = lax.axis_index("s")
    _prologue(src, dst, s, sidx, didx, zrows, acc, s, sga, sgb)

    @pl.when(c == 0)
    def _():
        _edge_chunks(tl, src, dst, s, sidx, didx, gbuf, acc, sga, sgb, NRB2)

    @pl.when(c == 1)
    def _():
        _edge_chunks(tr, src, dst, s, sidx, didx, gbuf, acc, sga, sgb, NRB2)

    plsc.subcore_barrier()
    pltpu.sync_copy(acc.at[pl.ds(s * ZROWS, ZROWS)],
                    out.at[c, pl.ds(s * ZROWS, ZROWS)])


def _relu_body(x_ref, o_ref):
    o_ref[...] = jnp.maximum(x_ref[...], 0.0)


def _mlp1_body(eps_ref, x_ref, p0_ref, p1_ref, wa_ref, ba_ref, wb_ref,
               bb_ref, ol_ref, or_ref):
    e = eps_ref[0, 0]
    t = (1.0 + e) * x_ref[...] + p0_ref[0] + p1_ref[0]
    a = jnp.maximum(
        jnp.dot(t, wa_ref[...], preferred_element_type=jnp.float32)
        + ba_ref[...], 0.0)
    h = jnp.maximum(
        jnp.dot(a, wb_ref[...], preferred_element_type=jnp.float32)
        + bb_ref[...], 0.0)
    ol_ref[...] = h[:, :128]
    or_ref[...] = h[:, 128:]


def _mlp2_body(eps_ref, hl_ref, hr_ref, a0_ref, a1_ref, b_ref, wa_ref,
               ba_ref, wb_ref, bb_ref, wm1_ref, bm1_ref, wm2_ref, bm2_ref,
               o_ref, pool_ref):
    i = pl.program_id(0)
    e = eps_ref[0, 0]
    h1 = jnp.concatenate([hl_ref[...], hr_ref[...]], axis=1)
    agg = jnp.concatenate([a0_ref[0], a1_ref[0]], axis=1)
    t = (1.0 + e) * h1 + agg
    a = jnp.maximum(
        jnp.dot(t, wa_ref[...], preferred_element_type=jnp.float32)
        + ba_ref[...], 0.0)
    h2 = jnp.dot(a, wb_ref[...], preferred_element_type=jnp.float32) + bb_ref[...]
    ids = b_ref[0, 0, :]
    oh = (lax.broadcasted_iota(jnp.int32, (64, MB), 0)
          == ids[None, :]).astype(jnp.float32)
    part = jnp.dot(oh, h2, preferred_element_type=jnp.float32)

    @pl.when(i == 0)
    def _():
        pool_ref[...] = part

    @pl.when(i > 0)
    def _():
        pool_ref[...] += part

    @pl.when(i == pl.num_programs(0) - 1)
    def _():
        q = jnp.maximum(
            jnp.dot(pool_ref[...], wm1_ref[...],
                    preferred_element_type=jnp.float32) + bm1_ref[...], 0.0)
        o_ref[...] = (jnp.dot(q, wm2_ref[...],
                              preferred_element_type=jnp.float32)
                      + bm2_ref[...])


def _full(shape):
    nd = len(shape)

    def im(i):
        return (0,) * nd

    return pl.BlockSpec(shape, im)


def kernel(x, edge_index, batch, eps1, W1a, b1a, W1b, b1b, eps2, W2a, b2a,
           W2b, b2b, Wm1, bm1, Wm2, bm2):
    src = edge_index[0].astype(jnp.int32)
    dst = edge_index[1].astype(jnp.int32)
    pad = EPAD - E
    ppos = jnp.arange(pad, dtype=jnp.int32)
    srcp = jnp.concatenate([src, (ppos * 37) % N])
    dstp = jnp.concatenate([dst, N + ppos % (NPAD - N)])
    src1 = srcp.reshape(32, NRB1, RB, K)
    dst1 = dstp.reshape(32, NRB1, RB, K)
    src2 = srcp.reshape(16, NRB2, RB, K)
    dst2 = dstp.reshape(16, NRB2, RB, K)
    zrows = jnp.zeros((ZROWS, 128), jnp.float32)
    batch3 = batch.astype(jnp.int32).reshape(2, 1, MB)
    e1 = jnp.reshape(eps1, (1, 1))
    e2 = jnp.reshape(eps2, (1, 1))
    b1a_ = b1a.reshape(1, 256)
    b1b_ = b1b.reshape(1, 256)
    b2a_ = b2a.reshape(1, 256)
    b2b_ = b2b.reshape(1, 256)
    bm1_ = bm1.reshape(1, 128)
    bm2_ = bm2.reshape(1, 2)

    xr = pl.pallas_call(
        _relu_body,
        grid=(2,),
        in_specs=[pl.BlockSpec((MB, 128), lambda i: (i, 0))],
        out_specs=pl.BlockSpec((MB, 128), lambda i: (i, 0)),
        out_shape=jax.ShapeDtypeStruct((N, 128), jnp.float32),
    )(x)

    p = _agg1(xr, src1, dst1, zrows)

    hl, hr = pl.pallas_call(
        _mlp1_body,
        grid=(2,),
        in_specs=[
            _full((1, 1)),
            pl.BlockSpec((MB, 128), lambda i: (i, 0)),
            pl.BlockSpec((1, MB, 128), lambda i: (0, i, 0)),
            pl.BlockSpec((1, MB, 128), lambda i: (1, i, 0)),
            _full((128, 256)),
            _full((1, 256)),
            _full((256, 256)),
            _full((1, 256)),
        ],
        out_specs=[
            pl.BlockSpec((MB, 128), lambda i: (i, 0)),
            pl.BlockSpec((MB, 128), lambda i: (i, 0)),
        ],
        out_shape=[
            jax.ShapeDtypeStruct((N, 128), jnp.float32),
            jax.ShapeDtypeStruct((N, 128), jnp.float32),
        ],
    )(e1, x, p, p, W1a, b1a_, W1b, b1b_)

    a2 = _agg2(hl, hr, src2, dst2, zrows)

    out = pl.pallas_call(
        _mlp2_body,
        grid=(2,),
        in_specs=[
            _full((1, 1)),
            pl.BlockSpec((MB, 128), lambda i: (i, 0)),
            pl.BlockSpec((MB, 128), lambda i: (i, 0)),
            pl.BlockSpec((1, MB, 128), lambda i: (0, i, 0)),
            pl.BlockSpec((1, MB, 128), lambda i: (1, i, 0)),
            pl.BlockSpec((1, 1, MB), lambda i: (i, 0, 0)),
            _full((256, 256)),
            _full((1, 256)),
            _full((256, 256)),
            _full((1, 256)),
            _full((256, 128)),
            _full((1, 128)),
            _full((128, 2)),
            _full((1, 2)),
        ],
        out_specs=pl.BlockSpec((64, 2), lambda i: (0, 0)),
        out_shape=jax.ShapeDtypeStruct((64, 2), jnp.float32),
        scratch_shapes=[pltpu.VMEM((64, 256), jnp.float32)],
    )(e2, hl, hr, a2, a2, batch3, W2a, b2a_, W2b, b2b_, Wm1, bm1_, Wm2, bm2_)

    return out
